# baseline (speedup 1.0000x reference)
import functools
import jax
import jax.numpy as jnp
from jax.experimental import pallas as pl
from jax.experimental.pallas import tpu as pltpu

LN_EPS = 1e-5
D_PAD = 128
TILE_E = 1024
CORE_SPLIT = 2


def _accumulate_kernel(x_ref, ea_ref, src_ref, dst_ref, m_ref, o_ref, *, d_edge):
    t = pl.program_id(1)

    @pl.when(t == 0)
    def _init():
        o_ref[...] = jnp.zeros_like(o_ref)

    tile_e = ea_ref.shape[0]
    n_pad = x_ref.shape[0]

    gid = jax.lax.broadcasted_iota(jnp.int32, (tile_e, n_pad), 1)
    oh_src = (src_ref[...] == gid).astype(jnp.bfloat16)
    g = jnp.dot(oh_src, x_ref[...],
                preferred_element_type=jnp.float32)

    ea = ea_ref[...]
    u = jnp.concatenate([ea[:, j:j + 1] * g for j in range(d_edge)],
                        axis=1).astype(jnp.bfloat16)

    msg = jnp.dot(u, m_ref[...], preferred_element_type=jnp.float32)

    sid = jax.lax.broadcasted_iota(jnp.int32, (n_pad, tile_e), 0)
    oh_dst = (sid == dst_ref[...]).astype(jnp.bfloat16)
    o_ref[...] += jnp.dot(oh_dst, msg.astype(jnp.bfloat16),
                          preferred_element_type=jnp.float32)


def _finalize_kernel(part_ref, ln_ref, o_ref, *, core_split, n_pad, d_out_real):
    agg = part_ref[0:n_pad, :]
    for c in range(1, core_split):
        agg = agg + part_ref[c * n_pad:(c + 1) * n_pad, :]

    lane = jax.lax.broadcasted_iota(jnp.int32, agg.shape, 1)
    mask = lane < d_out_real
    mu = jnp.sum(agg, axis=-1, keepdims=True) / d_out_real
    diff = agg - mu
    diff_m = jnp.where(mask, diff, 0.0)
    var = jnp.sum(diff_m * diff_m, axis=-1, keepdims=True) / d_out_real
    h = diff * jax.lax.rsqrt(var + LN_EPS)
    o_ref[...] = h * ln_ref[0:1, :] + ln_ref[1:2, :]


def kernel(node_features, edge_index, edge_attr, node_pos, C, W1, W2, gamma, beta):
    del node_pos
    N, d_in = node_features.shape
    E, d_edge = edge_attr.shape
    d_out = W2.shape[1]

    M = jnp.einsum('ijk,kh,ho->ijo', C, W1, W2)
    m_all = jnp.zeros((d_edge, D_PAD, D_PAD), jnp.float32)
    m_all = m_all.at[:, :d_in, :d_out].set(jnp.transpose(M, (1, 0, 2)))
    m_all = m_all.reshape(d_edge * D_PAD, D_PAD).astype(jnp.bfloat16)

    ln = jnp.zeros((2, D_PAD), jnp.float32)
    ln = ln.at[0, :d_out].set(gamma.astype(jnp.float32))
    ln = ln.at[1, :d_out].set(beta.astype(jnp.float32))

    n_pad = ((N + 7) // 8) * 8
    x = jnp.zeros((n_pad, D_PAD), jnp.float32)
    x = x.at[:N, :d_in].set(node_features.astype(jnp.float32))
    x = x.astype(jnp.bfloat16)

    chunk = CORE_SPLIT * TILE_E
    e_pad = ((E + chunk - 1) // chunk) * chunk
    steps = e_pad // chunk
    ea = jnp.zeros((e_pad, d_edge), jnp.float32).at[:E].set(edge_attr.astype(jnp.float32))
    src = jnp.zeros((e_pad, 1), jnp.int32).at[:E, 0].set(edge_index[0].astype(jnp.int32))
    dst = jnp.zeros((1, e_pad), jnp.int32).at[0, :E].set(edge_index[1].astype(jnp.int32))

    partial = pl.pallas_call(
        functools.partial(_accumulate_kernel, d_edge=d_edge),
        out_shape=jax.ShapeDtypeStruct((CORE_SPLIT * n_pad, D_PAD), jnp.float32),
        grid=(CORE_SPLIT, steps),
        in_specs=[
            pl.BlockSpec((n_pad, D_PAD), lambda c, t: (0, 0)),
            pl.BlockSpec((TILE_E, d_edge), lambda c, t: (c * steps + t, 0)),
            pl.BlockSpec((TILE_E, 1), lambda c, t: (c * steps + t, 0)),
            pl.BlockSpec((1, TILE_E), lambda c, t: (0, c * steps + t)),
            pl.BlockSpec((d_edge * D_PAD, D_PAD), lambda c, t: (0, 0)),
        ],
        out_specs=pl.BlockSpec((n_pad, D_PAD), lambda c, t: (c, 0)),
        compiler_params=pltpu.CompilerParams(
            dimension_semantics=("parallel", "arbitrary"),
            vmem_limit_bytes=48 * 1024 * 1024,
        ),
    )(x, ea, src, dst, m_all)

    out = pl.pallas_call(
        functools.partial(_finalize_kernel, core_split=CORE_SPLIT,
                          n_pad=n_pad, d_out_real=d_out),
        out_shape=jax.ShapeDtypeStruct((n_pad, D_PAD), jnp.float32),
        grid=(1,),
        in_specs=[
            pl.BlockSpec((CORE_SPLIT * n_pad, D_PAD), lambda i: (0, 0)),
            pl.BlockSpec((2, D_PAD), lambda i: (0, 0)),
        ],
        out_specs=pl.BlockSpec((n_pad, D_PAD), lambda i: (0, 0)),
    )(partial, ln)

    return out[:N, :d_out]


# speedup vs baseline: 1.0932x; 1.0932x over previous
import functools
import jax
import jax.numpy as jnp
from jax.experimental import pallas as pl
from jax.experimental.pallas import tpu as pltpu

LN_EPS = 1e-5
D_PAD = 128
TILE_E = 4096
CORE_SPLIT = 2


def _accumulate_kernel(x_ref, ea_ref, src_ref, dst_ref, m_ref, o_ref, *, d_edge):
    t = pl.program_id(1)

    @pl.when(t == 0)
    def _init():
        o_ref[...] = jnp.zeros_like(o_ref)

    tile_e = ea_ref.shape[0]
    n_pad = x_ref.shape[0]

    gid = jax.lax.broadcasted_iota(jnp.int32, (tile_e, n_pad), 1)
    oh_src = (src_ref[...] == gid).astype(jnp.bfloat16)
    g = jnp.dot(oh_src, x_ref[...],
                preferred_element_type=jnp.float32)

    ea = ea_ref[...]
    u = jnp.concatenate([ea[:, j:j + 1] * g for j in range(d_edge)],
                        axis=1).astype(jnp.bfloat16)

    msg = jnp.dot(u, m_ref[...], preferred_element_type=jnp.float32)

    sid = jax.lax.broadcasted_iota(jnp.int32, (n_pad, tile_e), 0)
    oh_dst = (sid == dst_ref[...]).astype(jnp.bfloat16)
    o_ref[...] += jnp.dot(oh_dst, msg.astype(jnp.bfloat16),
                          preferred_element_type=jnp.float32)


def _finalize_kernel(part_ref, ln_ref, o_ref, *, core_split, n_pad, d_out_real):
    agg = part_ref[0:n_pad, :]
    for c in range(1, core_split):
        agg = agg + part_ref[c * n_pad:(c + 1) * n_pad, :]

    lane = jax.lax.broadcasted_iota(jnp.int32, agg.shape, 1)
    mask = lane < d_out_real
    mu = jnp.sum(agg, axis=-1, keepdims=True) / d_out_real
    diff = agg - mu
    diff_m = jnp.where(mask, diff, 0.0)
    var = jnp.sum(diff_m * diff_m, axis=-1, keepdims=True) / d_out_real
    h = diff * jax.lax.rsqrt(var + LN_EPS)
    o_ref[...] = h * ln_ref[0:1, :] + ln_ref[1:2, :]


def kernel(node_features, edge_index, edge_attr, node_pos, C, W1, W2, gamma, beta):
    del node_pos
    N, d_in = node_features.shape
    E, d_edge = edge_attr.shape
    d_out = W2.shape[1]

    M = jnp.einsum('ijk,kh,ho->ijo', C, W1, W2)
    m_all = jnp.zeros((d_edge, D_PAD, D_PAD), jnp.float32)
    m_all = m_all.at[:, :d_in, :d_out].set(jnp.transpose(M, (1, 0, 2)))
    m_all = m_all.reshape(d_edge * D_PAD, D_PAD).astype(jnp.bfloat16)

    ln = jnp.zeros((2, D_PAD), jnp.float32)
    ln = ln.at[0, :d_out].set(gamma.astype(jnp.float32))
    ln = ln.at[1, :d_out].set(beta.astype(jnp.float32))

    n_pad = ((N + 7) // 8) * 8
    x = jnp.zeros((n_pad, D_PAD), jnp.float32)
    x = x.at[:N, :d_in].set(node_features.astype(jnp.float32))
    x = x.astype(jnp.bfloat16)

    chunk = CORE_SPLIT * TILE_E
    e_pad = ((E + chunk - 1) // chunk) * chunk
    steps = e_pad // chunk
    ea = jnp.zeros((e_pad, d_edge), jnp.float32).at[:E].set(edge_attr.astype(jnp.float32))
    src = jnp.zeros((e_pad, 1), jnp.int32).at[:E, 0].set(edge_index[0].astype(jnp.int32))
    dst = jnp.zeros((1, e_pad), jnp.int32).at[0, :E].set(edge_index[1].astype(jnp.int32))

    partial = pl.pallas_call(
        functools.partial(_accumulate_kernel, d_edge=d_edge),
        out_shape=jax.ShapeDtypeStruct((CORE_SPLIT * n_pad, D_PAD), jnp.float32),
        grid=(CORE_SPLIT, steps),
        in_specs=[
            pl.BlockSpec((n_pad, D_PAD), lambda c, t: (0, 0)),
            pl.BlockSpec((TILE_E, d_edge), lambda c, t: (c * steps + t, 0)),
            pl.BlockSpec((TILE_E, 1), lambda c, t: (c * steps + t, 0)),
            pl.BlockSpec((1, TILE_E), lambda c, t: (0, c * steps + t)),
            pl.BlockSpec((d_edge * D_PAD, D_PAD), lambda c, t: (0, 0)),
        ],
        out_specs=pl.BlockSpec((n_pad, D_PAD), lambda c, t: (c, 0)),
        compiler_params=pltpu.CompilerParams(
            dimension_semantics=("parallel", "arbitrary"),
            vmem_limit_bytes=48 * 1024 * 1024,
        ),
    )(x, ea, src, dst, m_all)

    out = pl.pallas_call(
        functools.partial(_finalize_kernel, core_split=CORE_SPLIT,
                          n_pad=n_pad, d_out_real=d_out),
        out_shape=jax.ShapeDtypeStruct((n_pad, D_PAD), jnp.float32),
        grid=(1,),
        in_specs=[
            pl.BlockSpec((CORE_SPLIT * n_pad, D_PAD), lambda i: (0, 0)),
            pl.BlockSpec((2, D_PAD), lambda i: (0, 0)),
        ],
        out_specs=pl.BlockSpec((n_pad, D_PAD), lambda i: (0, 0)),
    )(partial, ln)

    return out[:N, :d_out]


# speedup vs baseline: 1.2363x; 1.1309x over previous
import functools
import jax
import jax.numpy as jnp
from jax.experimental import pallas as pl
from jax.experimental.pallas import tpu as pltpu

LN_EPS = 1e-5
D_PAD = 128
TILE_E = 4096
CORE_SPLIT = 2


def _accumulate_kernel(x_ref, ea_ref, src_ref, dst_ref, m_ref, p_ref, o_ref, *, d_edge):
    t = pl.program_id(1)

    @pl.when(t == 0)
    def _init():
        o_ref[...] = jnp.zeros_like(o_ref)

    tile_e = ea_ref.shape[0]
    n_pad = x_ref.shape[0]

    gid = jax.lax.broadcasted_iota(jnp.int32, (tile_e, n_pad), 1)
    oh_src = (src_ref[...] == gid).astype(jnp.bfloat16)
    g = jnp.dot(oh_src, x_ref[...],
                preferred_element_type=jnp.float32)

    ea_exp = jnp.dot(ea_ref[...], p_ref[...],
                     preferred_element_type=jnp.float32)
    g_rep = jnp.concatenate([g] * d_edge, axis=1)
    u = (ea_exp * g_rep).astype(jnp.bfloat16)

    msg = jnp.dot(u, m_ref[...], preferred_element_type=jnp.float32)

    sid = jax.lax.broadcasted_iota(jnp.int32, (n_pad, tile_e), 0)
    oh_dst = (sid == dst_ref[...]).astype(jnp.bfloat16)
    o_ref[...] += jnp.dot(oh_dst, msg.astype(jnp.bfloat16),
                          preferred_element_type=jnp.float32)


def _finalize_kernel(part_ref, ln_ref, o_ref, *, core_split, n_pad, d_out_real):
    agg = part_ref[0:n_pad, :]
    for c in range(1, core_split):
        agg = agg + part_ref[c * n_pad:(c + 1) * n_pad, :]

    lane = jax.lax.broadcasted_iota(jnp.int32, agg.shape, 1)
    mask = lane < d_out_real
    mu = jnp.sum(agg, axis=-1, keepdims=True) / d_out_real
    diff = agg - mu
    diff_m = jnp.where(mask, diff, 0.0)
    var = jnp.sum(diff_m * diff_m, axis=-1, keepdims=True) / d_out_real
    h = diff * jax.lax.rsqrt(var + LN_EPS)
    o_ref[...] = h * ln_ref[0:1, :] + ln_ref[1:2, :]


def kernel(node_features, edge_index, edge_attr, node_pos, C, W1, W2, gamma, beta):
    del node_pos
    N, d_in = node_features.shape
    E, d_edge = edge_attr.shape
    d_out = W2.shape[1]

    M = jnp.einsum('ijk,kh,ho->ijo', C, W1, W2)
    m_all = jnp.zeros((d_edge, D_PAD, D_PAD), jnp.float32)
    m_all = m_all.at[:, :d_in, :d_out].set(jnp.transpose(M, (1, 0, 2)))
    m_all = m_all.reshape(d_edge * D_PAD, D_PAD).astype(jnp.bfloat16)

    ln = jnp.zeros((2, D_PAD), jnp.float32)
    ln = ln.at[0, :d_out].set(gamma.astype(jnp.float32))
    ln = ln.at[1, :d_out].set(beta.astype(jnp.float32))

    n_pad = ((N + 7) // 8) * 8
    x = jnp.zeros((n_pad, D_PAD), jnp.float32)
    x = x.at[:N, :d_in].set(node_features.astype(jnp.float32))
    x = x.astype(jnp.bfloat16)

    chunk = CORE_SPLIT * TILE_E
    e_pad = ((E + chunk - 1) // chunk) * chunk
    steps = e_pad // chunk
    ea = jnp.zeros((e_pad, d_edge), jnp.float32).at[:E].set(edge_attr.astype(jnp.float32))
    ea = ea.astype(jnp.bfloat16)
    p_exp = jnp.repeat(jnp.eye(d_edge, dtype=jnp.bfloat16), D_PAD, axis=1)
    src = jnp.zeros((e_pad, 1), jnp.int32).at[:E, 0].set(edge_index[0].astype(jnp.int32))
    dst = jnp.zeros((1, e_pad), jnp.int32).at[0, :E].set(edge_index[1].astype(jnp.int32))

    partial = pl.pallas_call(
        functools.partial(_accumulate_kernel, d_edge=d_edge),
        out_shape=jax.ShapeDtypeStruct((CORE_SPLIT * n_pad, D_PAD), jnp.float32),
        grid=(CORE_SPLIT, steps),
        in_specs=[
            pl.BlockSpec((n_pad, D_PAD), lambda c, t: (0, 0)),
            pl.BlockSpec((TILE_E, d_edge), lambda c, t: (c * steps + t, 0)),
            pl.BlockSpec((TILE_E, 1), lambda c, t: (c * steps + t, 0)),
            pl.BlockSpec((1, TILE_E), lambda c, t: (0, c * steps + t)),
            pl.BlockSpec((d_edge * D_PAD, D_PAD), lambda c, t: (0, 0)),
            pl.BlockSpec((d_edge, d_edge * D_PAD), lambda c, t: (0, 0)),
        ],
        out_specs=pl.BlockSpec((n_pad, D_PAD), lambda c, t: (c, 0)),
        compiler_params=pltpu.CompilerParams(
            dimension_semantics=("parallel", "arbitrary"),
            vmem_limit_bytes=48 * 1024 * 1024,
        ),
    )(x, ea, src, dst, m_all, p_exp)

    out = pl.pallas_call(
        functools.partial(_finalize_kernel, core_split=CORE_SPLIT,
                          n_pad=n_pad, d_out_real=d_out),
        out_shape=jax.ShapeDtypeStruct((n_pad, D_PAD), jnp.float32),
        grid=(1,),
        in_specs=[
            pl.BlockSpec((CORE_SPLIT * n_pad, D_PAD), lambda i: (0, 0)),
            pl.BlockSpec((2, D_PAD), lambda i: (0, 0)),
        ],
        out_specs=pl.BlockSpec((n_pad, D_PAD), lambda i: (0, 0)),
    )(partial, ln)

    return out[:N, :d_out]
